# baseline (speedup 1.0000x reference)
import functools

import jax
import jax.numpy as jnp
from jax import lax
from jax.experimental import pallas as pl
from jax.experimental.pallas import tpu as pltpu
from jax.experimental.pallas import tpu_sc as plsc

BATCH = 1024
SEQ = 200
EMB = 64
LANES = 128
NC = 2
NS = 16
NW = NC * NS

TOTAL = BATCH * SEQ
ROWS_PER_W = TOTAL // NW
CHUNK = 128
NCHUNK = ROWS_PER_W // CHUNK
IROWS_PER_W = ROWS_PER_W // LANES
IROWS_STRIDE = 56
GROUP = 8

def _build_embed_sc():
    mesh = plsc.VectorSubcoreMesh(
        core_axis_name="c", subcore_axis_name="s",
        num_cores=NC, num_subcores=NS,
    )
    return functools.partial(
        pl.kernel,
        out_type=jax.ShapeDtypeStruct((BATCH, SEQ, EMB), jnp.float32),
        mesh=mesh,
        scratch_types=[
        pltpu.VMEM((IROWS_STRIDE, LANES), jnp.int32),
        pltpu.VMEM((CHUNK, LANES), jnp.float32),
        pltpu.VMEM((CHUNK, LANES), jnp.float32),
        pltpu.VMEM((CHUNK, EMB), jnp.float32),
        pltpu.VMEM((CHUNK, EMB), jnp.float32),
        pltpu.VMEM((SEQ, LANES), jnp.float32),
        pltpu.SemaphoreType.DMA,
        pltpu.SemaphoreType.DMA,
        pltpu.SemaphoreType.DMA,
        pltpu.SemaphoreType.DMA,
        ],
    )


def _embed_sc_body(idx_hbm, tok_hbm, pos_hbm, out_hbm,
              idx_v, gath_a, gath_b, res_a, res_b, pos_v,
              gsem_a, gsem_b, wsem_a, wsem_b):
    wid = lax.axis_index("s") * NC + lax.axis_index("c")
    irow0 = wid * IROWS_STRIDE
    row0 = wid * ROWS_PER_W

    out2 = out_hbm.reshape(TOTAL, EMB)

    pltpu.sync_copy(idx_hbm.at[pl.ds(irow0, IROWS_STRIDE)], idx_v)
    pltpu.sync_copy(pos_hbm, pos_v)

    gath = (gath_a, gath_b)
    res = (res_a, res_b)
    gsem = (gsem_a, gsem_b)
    wsem = (wsem_a, wsem_b)

    def start_gather(b, cc):
        pltpu.async_copy(tok_hbm.at[idx_v.at[cc]], gath[b], gsem[b])

    def wait_gather(b):
        pltpu.make_async_copy(
            tok_hbm.at[pl.ds(0, CHUNK)], gath[b], gsem[b]).wait()

    def compute(b, cc):
        g, o = gath[b], res[b]
        s0 = lax.rem(cc * CHUNK, SEQ)

        @plsc.parallel_loop(0, CHUNK // GROUP, unroll=1, carry=s0)
        def body(t, s):
            for r in range(GROUP):
                j = GROUP * t + r
                se = s + r
                se = jnp.where(se >= SEQ, se - SEQ, se)
                for k in range(EMB // 16):
                    sl = pl.ds(k * 16, 16)
                    o[j, sl] = g[j, sl] + pos_v[se, sl]
            s = s + GROUP
            return jnp.where(s >= SEQ, s - SEQ, s)

    start_gather(0, 0)
    start_gather(1, 1)

    @pl.loop(0, NCHUNK, step=2)
    def chunk_pair(c):
        for b in range(2):
            cc = c + b
            wait_gather(b)
            compute(b, cc)
            w = pltpu.async_copy(
                res[b],
                out2.at[pl.ds(row0 + cc * CHUNK, CHUNK)],
                wsem[b],
            )
            w.wait()

            @pl.when(cc + 2 < NCHUNK)
            def _():
                start_gather(b, cc + 2)


_embed_sc_cache = []


def _embed_sc(idx, tok, pos):
    if not _embed_sc_cache:
        _embed_sc_cache.append(_build_embed_sc()(_embed_sc_body))
    return _embed_sc_cache[0](idx, tok, pos)


def _worker_blocks(x):
    x = x.reshape(NW, IROWS_PER_W, LANES)
    x = jnp.pad(x, ((0, 0), (0, IROWS_STRIDE - IROWS_PER_W), (0, 0)))
    return x.reshape(NW * IROWS_STRIDE, LANES)


def kernel(inputs, token_table, position_table):
    flat = inputs.reshape(-1).astype(jnp.int32)
    idx = _worker_blocks(flat)
    tokp = jnp.zeros((token_table.shape[0], LANES), jnp.float32)
    tokp = jax.lax.dynamic_update_slice(tokp, token_table, (0, 0))
    posp = jnp.pad(position_table, ((0, 0), (0, LANES - EMB)))
    return _embed_sc(idx, tokp, posp)


# speedup vs baseline: 1.1960x; 1.1960x over previous
import functools

import jax
import jax.numpy as jnp
from jax import lax
from jax.experimental import pallas as pl
from jax.experimental.pallas import tpu as pltpu
from jax.experimental.pallas import tpu_sc as plsc

BATCH = 1024
SEQ = 200
EMB = 64
LANES = 128
NC = 2
NS = 16
NW = NC * NS

TOTAL = BATCH * SEQ
ROWS_PER_W = TOTAL // NW
CHUNK = 128
NCHUNK = ROWS_PER_W // CHUNK
IROWS_PER_W = ROWS_PER_W // LANES
IROWS_STRIDE = 56
GROUP = 8

def _build_embed_sc():
    mesh = plsc.VectorSubcoreMesh(
        core_axis_name="c", subcore_axis_name="s",
        num_cores=NC, num_subcores=NS,
    )
    return functools.partial(
        pl.kernel,
        out_type=jax.ShapeDtypeStruct((BATCH, SEQ, EMB), jnp.float32),
        mesh=mesh,
        scratch_types=[
        pltpu.VMEM((IROWS_STRIDE, LANES), jnp.int32),
        pltpu.VMEM((CHUNK, LANES), jnp.float32),
        pltpu.VMEM((CHUNK, LANES), jnp.float32),
        pltpu.VMEM((CHUNK, EMB), jnp.float32),
        pltpu.VMEM((CHUNK, EMB), jnp.float32),
        pltpu.VMEM((SEQ, LANES), jnp.float32),
        pltpu.SemaphoreType.DMA,
        pltpu.SemaphoreType.DMA,
        pltpu.SemaphoreType.DMA,
        pltpu.SemaphoreType.DMA,
        ],
    )


def _embed_sc_body(idx_hbm, tok_hbm, pos_hbm, out_hbm,
              idx_v, gath_a, gath_b, res_a, res_b, pos_v,
              gsem_a, gsem_b, wsem_a, wsem_b):
    wid = lax.axis_index("s") * NC + lax.axis_index("c")
    irow0 = wid * IROWS_STRIDE
    row0 = wid * ROWS_PER_W

    out2 = out_hbm.reshape(TOTAL, EMB)

    pltpu.sync_copy(idx_hbm.at[pl.ds(irow0, IROWS_STRIDE)], idx_v)
    pltpu.sync_copy(pos_hbm, pos_v)

    gath = (gath_a, gath_b)
    res = (res_a, res_b)
    gsem = (gsem_a, gsem_b)
    wsem = (wsem_a, wsem_b)

    def start_gather(b, cc):
        pltpu.async_copy(tok_hbm.at[idx_v.at[cc]], gath[b], gsem[b])

    def wait_gather(b):
        pltpu.make_async_copy(
            tok_hbm.at[pl.ds(0, CHUNK)], gath[b], gsem[b]).wait()

    def compute(b, cc):
        g, o = gath[b], res[b]
        s0 = lax.rem(cc * CHUNK, SEQ)

        @plsc.parallel_loop(0, CHUNK // GROUP, unroll=1, carry=s0)
        def body(t, s):
            for r in range(GROUP):
                j = GROUP * t + r
                se = s + r
                se = jnp.where(se >= SEQ, se - SEQ, se)
                for k in range(EMB // 16):
                    sl = pl.ds(k * 16, 16)
                    o[j, sl] = g[j, sl] + pos_v[se, sl]
            s = s + GROUP
            return jnp.where(s >= SEQ, s - SEQ, s)

    start_gather(0, 0)
    start_gather(1, 1)

    @pl.loop(0, NCHUNK, step=2)
    def chunk_pair(c):
        for b in range(2):
            cc = c + b
            wait_gather(b)
            compute(b, cc)
            w = pltpu.async_copy(
                res[b],
                out2.at[pl.ds(row0 + cc * CHUNK, CHUNK)],
                wsem[b],
            )
            w.wait()

            @pl.when(cc + 2 < NCHUNK)
            def _():
                start_gather(b, cc + 2)


_embed_sc_cache = []


def _embed_sc(idx, tok, pos):
    if not _embed_sc_cache:
        _embed_sc_cache.append(_build_embed_sc()(_embed_sc_body))
    return _embed_sc_cache[0](idx, tok, pos)


def _worker_blocks(x):
    x = x.reshape(NW, IROWS_PER_W, LANES)
    x = jnp.pad(x, ((0, 0), (0, IROWS_STRIDE - IROWS_PER_W), (0, 0)))
    return x.reshape(NW * IROWS_STRIDE, LANES)


def kernel(inputs, token_table, position_table):
    flat = inputs.reshape(-1).astype(jnp.int32)
    idx = _worker_blocks(flat)
    widen = jnp.eye(EMB, LANES, dtype=jnp.float32)
    tokp = jax.lax.dot(token_table, widen,
                       precision=jax.lax.Precision.HIGHEST)
    posp = jnp.pad(position_table, ((0, 0), (0, LANES - EMB)))
    return _embed_sc(idx, tokp, posp)
